# baseline (speedup 1.0000x reference)
import functools

import jax
import jax.numpy as jnp
from jax.experimental import pallas as pl
from jax.experimental.pallas import tpu as pltpu
from jax.experimental.pallas import tpu_sc as plsc

_D = 128


def _dot_body(inputs_hbm, idx_hbm, table_hbm, out_hbm, idx_s, row_s, in_s,
              out_s, sem, sem2):
    cp_idx = pltpu.make_async_copy(idx_hbm, idx_s, sem2)
    cp_idx.start()
    pltpu.sync_copy(inputs_hbm, in_s)
    cp_idx.wait()
    idx = idx_s[0]
    pltpu.async_copy(table_hbm.at[idx], row_s, sem).wait()
    acc = [jnp.float32(0.0)] * 4
    for i in range(_D):
        acc[i % 4] = acc[i % 4] + row_s[i] * in_s[i]
    out_s[0] = (acc[0] + acc[1]) + (acc[2] + acc[3])
    pltpu.sync_copy(out_s, out_hbm)


@jax.jit
def _run(inputs, user_idx, table):
    mesh = plsc.ScalarSubcoreMesh(axis_name="c", num_cores=1)
    k = functools.partial(
        pl.kernel,
        mesh=mesh,
        out_type=jax.ShapeDtypeStruct((1,), jnp.float32),
        scratch_types=[
            pltpu.SMEM((1,), jnp.int32),
            pltpu.SMEM((_D,), jnp.float32),
            pltpu.SMEM((_D,), jnp.float32),
            pltpu.SMEM((1,), jnp.float32),
            pltpu.SemaphoreType.DMA,
            pltpu.SemaphoreType.DMA,
        ],
    )(_dot_body)
    out = k(inputs, user_idx.astype(jnp.int32), table)
    return out[0]


def kernel(inputs, user_idx, table):
    return _run(inputs, user_idx, table)
